# Initial kernel scaffold; baseline (speedup 1.0000x reference)
import jax
import jax.numpy as jnp
from jax.experimental import pallas as pl


def kernel(node_features, edge_features, from_idx, to_idx, graph_idx, Wn, bn, We, be, Wm1, bm1, Wm2, bm2, Wu1, bu1, Wu2, bu2, Wg, bg, Wf, bf):
    raise NotImplementedError("write your pallas kernel here")


# speedup vs baseline: 2.2149x; 2.2149x over previous
import jax
import jax.numpy as jnp
from jax import lax
from jax.experimental import pallas as pl
from jax.experimental.pallas import tpu as pltpu
from jax.experimental.pallas import tpu_sc as plsc

_N = 10000
_E = 320000
_G = 100
_NPAIRS = 50
_D = 128
_DE = 16
_DEH = 64
_HID = 256
_NPROP = 2

_NC = 2
_NS = 16
_NW = _NC * _NS
_EPW = _E // _NW
_KCH = 80
_NCH = _EPW // _KCH
_RPT = 640
_RLAST = _N - 15 * _RPT

_sc_mesh = plsc.VectorSubcoreMesh(core_axis_name="c", subcore_axis_name="s")


def _gather_body(h_hbm, fi_hbm, ti_hbm, fs_hbm, ts_hbm,
                 fidx_v, tidx_v, frow_v, trow_v, fsem, tsem):
    c = lax.axis_index("c")
    s = lax.axis_index("s")
    base = (s * _NC + c) * _EPW

    def chunk(i, carry):
        off = pl.multiple_of(base + i * _KCH, 8)
        pltpu.sync_copy(fi_hbm.at[pl.ds(off, _KCH)], fidx_v)
        pltpu.sync_copy(ti_hbm.at[pl.ds(off, _KCH)], tidx_v)
        cf = pltpu.async_copy(h_hbm.at[fidx_v], frow_v, fsem)
        ct = pltpu.async_copy(h_hbm.at[tidx_v], trow_v, tsem)
        cf.wait()
        ct.wait()
        pltpu.sync_copy(frow_v, fs_hbm.at[pl.ds(off, _KCH)])
        pltpu.sync_copy(trow_v, ts_hbm.at[pl.ds(off, _KCH)])
        return carry

    lax.fori_loop(0, _NCH, chunk, 0)


_gather = pl.kernel(
    _gather_body,
    out_type=[jax.ShapeDtypeStruct((_E, _D), jnp.float32),
              jax.ShapeDtypeStruct((_E, _D), jnp.float32)],
    mesh=_sc_mesh,
    scratch_types=[
        pltpu.VMEM((_KCH,), jnp.int32),
        pltpu.VMEM((_KCH,), jnp.int32),
        pltpu.VMEM((_KCH, _D), jnp.float32),
        pltpu.VMEM((_KCH, _D), jnp.float32),
        pltpu.SemaphoreType.DMA,
        pltpu.SemaphoreType.DMA,
    ],
)


def _scatter_body(m_hbm, ti_hbm, z_hbm, part_hbm, idx_v, row_v, acc_sh):
    c = lax.axis_index("c")
    s = lax.axis_index("s")
    r0 = pl.multiple_of(s * _RPT, 8)

    @pl.when(s < _NS - 1)
    def _zero_main():
        pltpu.sync_copy(z_hbm.at[pl.ds(r0, _RPT)], acc_sh.at[pl.ds(r0, _RPT)])

    @pl.when(s == _NS - 1)
    def _zero_last():
        pltpu.sync_copy(z_hbm.at[pl.ds(15 * _RPT, _RLAST)],
                        acc_sh.at[pl.ds(15 * _RPT, _RLAST)])

    plsc.subcore_barrier()
    base = c * (_E // _NC) + s * _EPW

    def chunk(i, carry):
        off = pl.multiple_of(base + i * _KCH, 8)
        pltpu.sync_copy(ti_hbm.at[pl.ds(off, _KCH)], idx_v)
        pltpu.sync_copy(m_hbm.at[pl.ds(off, _KCH)], row_v)
        pltpu.sync_copy(row_v, acc_sh.at[idx_v], add=True)
        return carry

    lax.fori_loop(0, _NCH, chunk, 0)
    plsc.subcore_barrier()

    @pl.when(s < _NS - 1)
    def _dump_main():
        pltpu.sync_copy(acc_sh.at[pl.ds(r0, _RPT)],
                        part_hbm.at[c, pl.ds(r0, _RPT)])

    @pl.when(s == _NS - 1)
    def _dump_last():
        pltpu.sync_copy(acc_sh.at[pl.ds(15 * _RPT, _RLAST)],
                        part_hbm.at[c, pl.ds(15 * _RPT, _RLAST)])


_scatter = pl.kernel(
    _scatter_body,
    out_type=jax.ShapeDtypeStruct((_NC, _N, _D), jnp.float32),
    mesh=_sc_mesh,
    scratch_types=[
        pltpu.VMEM((_KCH,), jnp.int32),
        pltpu.VMEM((_KCH, _D), jnp.float32),
        pltpu.VMEM_SHARED((_N, _D), jnp.float32),
    ],
)


def _enc_body(nf, Wn, bn, out):
    out[...] = jnp.maximum(nf[...] @ Wn[...] + bn[...], 0.0)


def _encode_h(nf, Wn, bn2):
    blk = 1000
    return pl.pallas_call(
        _enc_body,
        grid=(_N // blk,),
        in_specs=[pl.BlockSpec((blk, _D), lambda i: (i, 0)),
                  pl.BlockSpec((_D, _D), lambda i: (0, 0)),
                  pl.BlockSpec((1, _D), lambda i: (0, 0))],
        out_specs=pl.BlockSpec((blk, _D), lambda i: (i, 0)),
        out_shape=jax.ShapeDtypeStruct((_N, _D), jnp.float32),
    )(nf, Wn, bn2)


def _msg_body(fs, ts, ef, A, B, C, We, be, bm1, Wm2, bm2, out):
    e = jnp.maximum(ef[...] @ We[...] + be[...], 0.0)
    m1 = fs[...] @ A[...] + ts[...] @ B[...] + e @ C[...] + bm1[...]
    out[...] = jnp.maximum(m1, 0.0) @ Wm2[...] + bm2[...]


def _messages(fs, ts, ef, A, B, C, We, be2, bm12, Wm2, bm22):
    blk = 2000
    return pl.pallas_call(
        _msg_body,
        grid=(_E // blk,),
        in_specs=[pl.BlockSpec((blk, _D), lambda i: (i, 0)),
                  pl.BlockSpec((blk, _D), lambda i: (i, 0)),
                  pl.BlockSpec((blk, _DE), lambda i: (i, 0)),
                  pl.BlockSpec((_D, _HID), lambda i: (0, 0)),
                  pl.BlockSpec((_D, _HID), lambda i: (0, 0)),
                  pl.BlockSpec((_DEH, _HID), lambda i: (0, 0)),
                  pl.BlockSpec((_DE, _DEH), lambda i: (0, 0)),
                  pl.BlockSpec((1, _DEH), lambda i: (0, 0)),
                  pl.BlockSpec((1, _HID), lambda i: (0, 0)),
                  pl.BlockSpec((_HID, _D), lambda i: (0, 0)),
                  pl.BlockSpec((1, _D), lambda i: (0, 0))],
        out_specs=pl.BlockSpec((blk, _D), lambda i: (i, 0)),
        out_shape=jax.ShapeDtypeStruct((_E, _D), jnp.float32),
    )(fs, ts, ef, A, B, C, We, be2, bm12, Wm2, bm22)


def _rsm(x):
    m = jnp.max(x, axis=1, keepdims=True)
    ex = jnp.exp(x - m)
    return ex / jnp.sum(ex, axis=1, keepdims=True)


def _upd_body(hb, pb, U1h, U1a, U1c, bu1, Wu2, bu2, out):
    h = hb[...]
    p = pb[...]
    agg = p[0] + p[1]
    a = h[:_G]
    b = h[_G:]
    dnt = (((1,), (1,)), ((), ()))
    sim = lax.dot_general(a, b, dnt)
    simt = lax.dot_general(b, a, dnt)
    att_a = _rsm(sim) @ b
    att_b = _rsm(simt) @ a
    cross = jnp.concatenate([a - att_a, b - att_b], axis=0)
    u = jnp.maximum(h @ U1h[...] + agg @ U1a[...] + cross @ U1c[...]
                    + bu1[...], 0.0)
    out[...] = u @ Wu2[...] + bu2[...]


def _update(h, parts, U1h, U1a, U1c, bu12, Wu2, bu22):
    blk = 2 * _G
    return pl.pallas_call(
        _upd_body,
        grid=(_NPAIRS,),
        in_specs=[pl.BlockSpec((blk, _D), lambda i: (i, 0)),
                  pl.BlockSpec((_NC, blk, _D), lambda i: (0, i, 0)),
                  pl.BlockSpec((_D, _HID), lambda i: (0, 0)),
                  pl.BlockSpec((_D, _HID), lambda i: (0, 0)),
                  pl.BlockSpec((_D, _HID), lambda i: (0, 0)),
                  pl.BlockSpec((1, _HID), lambda i: (0, 0)),
                  pl.BlockSpec((_HID, _D), lambda i: (0, 0)),
                  pl.BlockSpec((1, _D), lambda i: (0, 0))],
        out_specs=pl.BlockSpec((blk, _D), lambda i: (i, 0)),
        out_shape=jax.ShapeDtypeStruct((_N, _D), jnp.float32),
    )(h, parts, U1h, U1a, U1c, bu12, Wu2, bu22)


def _pool_body(hb, Wg, bg, Wf, bf, out):
    gv = hb[...] @ Wg[...] + bg[...]
    gates = 1.0 / (1.0 + jnp.exp(-gv[:, :_D]))
    gated = gates * gv[:, _D:]
    ga = jnp.sum(gated[:_G], axis=0, keepdims=True)
    gb = jnp.sum(gated[_G:], axis=0, keepdims=True)
    va = ga @ Wf[...] + bf[...]
    vb = gb @ Wf[...] + bf[...]
    r = jnp.maximum(va - vb, 0.0)
    sc = -jnp.sum(r, axis=1, keepdims=True)
    out[...] = jnp.broadcast_to(sc.reshape(1, 1, 1), (1, 1, _D))


def _pool(h, Wg, bg2, Wf, bf2):
    blk = 2 * _G
    return pl.pallas_call(
        _pool_body,
        grid=(_NPAIRS,),
        in_specs=[pl.BlockSpec((blk, _D), lambda i: (i, 0)),
                  pl.BlockSpec((_D, 2 * _D), lambda i: (0, 0)),
                  pl.BlockSpec((1, 2 * _D), lambda i: (0, 0)),
                  pl.BlockSpec((_D, _D), lambda i: (0, 0)),
                  pl.BlockSpec((1, _D), lambda i: (0, 0))],
        out_specs=pl.BlockSpec((1, 1, _D), lambda i: (i, 0, 0)),
        out_shape=jax.ShapeDtypeStruct((_NPAIRS, 1, _D), jnp.float32),
    )(h, Wg, bg2, Wf, bf2)


def kernel(node_features, edge_features, from_idx, to_idx, graph_idx,
           Wn, bn, We, be, Wm1, bm1, Wm2, bm2, Wu1, bu1, Wu2, bu2,
           Wg, bg, Wf, bf):
    bn2 = bn.reshape(1, -1)
    be2 = be.reshape(1, -1)
    bm12 = bm1.reshape(1, -1)
    bm22 = bm2.reshape(1, -1)
    bu12 = bu1.reshape(1, -1)
    bu22 = bu2.reshape(1, -1)
    bg2 = bg.reshape(1, -1)
    bf2 = bf.reshape(1, -1)
    A = Wm1[:_D]
    B = Wm1[_D:2 * _D]
    C = Wm1[2 * _D:]
    U1h = Wu1[:_D]
    U1a = Wu1[_D:2 * _D]
    U1c = Wu1[2 * _D:]
    fi = from_idx.astype(jnp.int32)
    ti = to_idx.astype(jnp.int32)
    zeros = jnp.zeros((_N, _D), jnp.float32)

    h = _encode_h(node_features, Wn, bn2)
    for _ in range(_NPROP):
        fs, ts = _gather(h, fi, ti)
        m = _messages(fs, ts, edge_features, A, B, C, We, be2, bm12,
                      Wm2, bm22)
        parts = _scatter(m, ti, zeros)
        h = _update(h, parts, U1h, U1a, U1c, bu12, Wu2, bu22)
    out = _pool(h, Wg, bg2, Wf, bf2)
    return out[:, 0, 0]


# speedup vs baseline: 2.8395x; 1.2820x over previous
import jax
import jax.numpy as jnp
from jax import lax
from jax.experimental import pallas as pl
from jax.experimental.pallas import tpu as pltpu
from jax.experimental.pallas import tpu_sc as plsc

_N = 10000
_E = 320000
_G = 100
_NPAIRS = 50
_D = 128
_DE = 16
_DEH = 64
_HID = 256
_NPROP = 2

_NC = 2
_NS = 16
_NW = _NC * _NS
_EPW = _E // _NW
_KCH = 80
_NCH = _EPW // _KCH
_RPT = 640
_RLAST = _N - 15 * _RPT

_sc_mesh = plsc.VectorSubcoreMesh(core_axis_name="c", subcore_axis_name="s")


_GEPW = _E // _NS
_CPW = _GEPW // _KCH
_CPG = 2
_GRP = _CPG * _KCH
_NGRP = _CPW // _CPG


def _gather_body(h_hbm, idxg_hbm, st_hbm, idx_v, rows_v, gsem, wsem):
    c = lax.axis_index("c")
    s = lax.axis_index("s")
    pltpu.sync_copy(idxg_hbm.at[c, s], idx_v)
    ebase = s * _GEPW

    def group(g, carry):
        b = lax.rem(g, 2)

        @pl.when(g >= 2)
        def _drain():
            pltpu.make_async_copy(
                rows_v.at[b], st_hbm.at[c, pl.ds(0, _GRP)], wsem).wait()

        cps = [
            pltpu.async_copy(h_hbm.at[idx_v.at[g * _CPG + j]],
                             rows_v.at[b, pl.ds(j * _KCH, _KCH)], gsem)
            for j in range(_CPG)
        ]
        for cp in cps:
            cp.wait()
        off = pl.multiple_of(ebase + g * _GRP, 8)
        pltpu.async_copy(rows_v.at[b], st_hbm.at[c, pl.ds(off, _GRP)], wsem)
        return carry

    lax.fori_loop(0, _NGRP, group, 0)
    pltpu.make_async_copy(rows_v.at[0], st_hbm.at[c, pl.ds(0, _GRP)],
                          wsem).wait()
    pltpu.make_async_copy(rows_v.at[1], st_hbm.at[c, pl.ds(0, _GRP)],
                          wsem).wait()


_gather = pl.kernel(
    _gather_body,
    out_type=jax.ShapeDtypeStruct((_NC, _E, _D), jnp.float32),
    mesh=_sc_mesh,
    scratch_types=[
        pltpu.VMEM((_CPW, _KCH), jnp.int32),
        pltpu.VMEM((2, _GRP, _D), jnp.float32),
        pltpu.SemaphoreType.DMA,
        pltpu.SemaphoreType.DMA,
    ],
)


_SCH = _EPW // _KCH
_SNG = _SCH // _CPG


def _scatter_body(m_hbm, idxs_hbm, z_hbm, part_hbm, idx_v, mrows_v, msem,
                  acc_sh):
    c = lax.axis_index("c")
    s = lax.axis_index("s")
    r0 = pl.multiple_of(s * _RPT, 8)

    @pl.when(s < _NS - 1)
    def _zero_main():
        pltpu.sync_copy(z_hbm.at[pl.ds(r0, _RPT)], acc_sh.at[pl.ds(r0, _RPT)])

    @pl.when(s == _NS - 1)
    def _zero_last():
        pltpu.sync_copy(z_hbm.at[pl.ds(15 * _RPT, _RLAST)],
                        acc_sh.at[pl.ds(15 * _RPT, _RLAST)])

    pltpu.sync_copy(idxs_hbm.at[c, s], idx_v)
    plsc.subcore_barrier()
    ebase = c * (_E // _NC) + s * _EPW
    pltpu.async_copy(m_hbm.at[pl.ds(ebase, _KCH)], mrows_v.at[0], msem)

    def chunk(g, carry):
        b = lax.rem(g, 2)
        pltpu.make_async_copy(m_hbm.at[pl.ds(0, _KCH)], mrows_v.at[b],
                              msem).wait()

        @pl.when(g < _SCH - 1)
        def _prefetch():
            off = pl.multiple_of(ebase + (g + 1) * _KCH, 8)
            pltpu.async_copy(m_hbm.at[pl.ds(off, _KCH)],
                             mrows_v.at[lax.rem(g + 1, 2)], msem)

        pltpu.sync_copy(mrows_v.at[b], acc_sh.at[idx_v.at[g]], add=True)
        return carry

    lax.fori_loop(0, _SCH, chunk, 0)
    plsc.subcore_barrier()

    @pl.when(s < _NS - 1)
    def _dump_main():
        pltpu.sync_copy(acc_sh.at[pl.ds(r0, _RPT)],
                        part_hbm.at[c, pl.ds(r0, _RPT)])

    @pl.when(s == _NS - 1)
    def _dump_last():
        pltpu.sync_copy(acc_sh.at[pl.ds(15 * _RPT, _RLAST)],
                        part_hbm.at[c, pl.ds(15 * _RPT, _RLAST)])


_scatter = pl.kernel(
    _scatter_body,
    out_type=jax.ShapeDtypeStruct((_NC, _N, _D), jnp.float32),
    mesh=_sc_mesh,
    scratch_types=[
        pltpu.VMEM((_SCH, _KCH), jnp.int32),
        pltpu.VMEM((2, _KCH, _D), jnp.float32),
        pltpu.SemaphoreType.DMA,
        pltpu.VMEM_SHARED((_N, _D), jnp.float32),
    ],
)


def _enc_body(nf, Wn, bn, out):
    out[...] = jnp.maximum(nf[...] @ Wn[...] + bn[...], 0.0)


def _encode_h(nf, Wn, bn2):
    blk = 1000
    return pl.pallas_call(
        _enc_body,
        grid=(_N // blk,),
        in_specs=[pl.BlockSpec((blk, _D), lambda i: (i, 0)),
                  pl.BlockSpec((_D, _D), lambda i: (0, 0)),
                  pl.BlockSpec((1, _D), lambda i: (0, 0))],
        out_specs=pl.BlockSpec((blk, _D), lambda i: (i, 0)),
        out_shape=jax.ShapeDtypeStruct((_N, _D), jnp.float32),
    )(nf, Wn, bn2)


def _msg_body(st, ef, A, B, C, We, be, bm1, Wm2, bm2, out):
    e = jnp.maximum(ef[...] @ We[...] + be[...], 0.0)
    stv = st[...]
    m1 = stv[0] @ A[...] + stv[1] @ B[...] + e @ C[...] + bm1[...]
    out[...] = jnp.maximum(m1, 0.0) @ Wm2[...] + bm2[...]


def _messages(st, ef, A, B, C, We, be2, bm12, Wm2, bm22):
    blk = 2000
    return pl.pallas_call(
        _msg_body,
        grid=(_E // blk,),
        in_specs=[pl.BlockSpec((_NC, blk, _D), lambda i: (0, i, 0)),
                  pl.BlockSpec((blk, _DE), lambda i: (i, 0)),
                  pl.BlockSpec((_D, _HID), lambda i: (0, 0)),
                  pl.BlockSpec((_D, _HID), lambda i: (0, 0)),
                  pl.BlockSpec((_DEH, _HID), lambda i: (0, 0)),
                  pl.BlockSpec((_DE, _DEH), lambda i: (0, 0)),
                  pl.BlockSpec((1, _DEH), lambda i: (0, 0)),
                  pl.BlockSpec((1, _HID), lambda i: (0, 0)),
                  pl.BlockSpec((_HID, _D), lambda i: (0, 0)),
                  pl.BlockSpec((1, _D), lambda i: (0, 0))],
        out_specs=pl.BlockSpec((blk, _D), lambda i: (i, 0)),
        out_shape=jax.ShapeDtypeStruct((_E, _D), jnp.float32),
    )(st, ef, A, B, C, We, be2, bm12, Wm2, bm22)


def _rsm(x):
    m = jnp.max(x, axis=1, keepdims=True)
    ex = jnp.exp(x - m)
    return ex / jnp.sum(ex, axis=1, keepdims=True)


def _upd_body(hb, pb, U1h, U1a, U1c, bu1, Wu2, bu2, out):
    h = hb[...]
    p = pb[...]
    agg = p[0] + p[1]
    a = h[:_G]
    b = h[_G:]
    dnt = (((1,), (1,)), ((), ()))
    sim = lax.dot_general(a, b, dnt)
    simt = lax.dot_general(b, a, dnt)
    att_a = _rsm(sim) @ b
    att_b = _rsm(simt) @ a
    cross = jnp.concatenate([a - att_a, b - att_b], axis=0)
    u = jnp.maximum(h @ U1h[...] + agg @ U1a[...] + cross @ U1c[...]
                    + bu1[...], 0.0)
    out[...] = u @ Wu2[...] + bu2[...]


def _update(h, parts, U1h, U1a, U1c, bu12, Wu2, bu22):
    blk = 2 * _G
    return pl.pallas_call(
        _upd_body,
        grid=(_NPAIRS,),
        in_specs=[pl.BlockSpec((blk, _D), lambda i: (i, 0)),
                  pl.BlockSpec((_NC, blk, _D), lambda i: (0, i, 0)),
                  pl.BlockSpec((_D, _HID), lambda i: (0, 0)),
                  pl.BlockSpec((_D, _HID), lambda i: (0, 0)),
                  pl.BlockSpec((_D, _HID), lambda i: (0, 0)),
                  pl.BlockSpec((1, _HID), lambda i: (0, 0)),
                  pl.BlockSpec((_HID, _D), lambda i: (0, 0)),
                  pl.BlockSpec((1, _D), lambda i: (0, 0))],
        out_specs=pl.BlockSpec((blk, _D), lambda i: (i, 0)),
        out_shape=jax.ShapeDtypeStruct((_N, _D), jnp.float32),
    )(h, parts, U1h, U1a, U1c, bu12, Wu2, bu22)


def _pool_body(hb, Wg, bg, Wf, bf, out):
    gv = hb[...] @ Wg[...] + bg[...]
    gates = 1.0 / (1.0 + jnp.exp(-gv[:, :_D]))
    gated = gates * gv[:, _D:]
    ga = jnp.sum(gated[:_G], axis=0, keepdims=True)
    gb = jnp.sum(gated[_G:], axis=0, keepdims=True)
    va = ga @ Wf[...] + bf[...]
    vb = gb @ Wf[...] + bf[...]
    r = jnp.maximum(va - vb, 0.0)
    sc = -jnp.sum(r, axis=1, keepdims=True)
    out[...] = jnp.broadcast_to(sc.reshape(1, 1, 1), (1, 1, _D))


def _pool(h, Wg, bg2, Wf, bf2):
    blk = 2 * _G
    return pl.pallas_call(
        _pool_body,
        grid=(_NPAIRS,),
        in_specs=[pl.BlockSpec((blk, _D), lambda i: (i, 0)),
                  pl.BlockSpec((_D, 2 * _D), lambda i: (0, 0)),
                  pl.BlockSpec((1, 2 * _D), lambda i: (0, 0)),
                  pl.BlockSpec((_D, _D), lambda i: (0, 0)),
                  pl.BlockSpec((1, _D), lambda i: (0, 0))],
        out_specs=pl.BlockSpec((1, 1, _D), lambda i: (i, 0, 0)),
        out_shape=jax.ShapeDtypeStruct((_NPAIRS, 1, _D), jnp.float32),
    )(h, Wg, bg2, Wf, bf2)


def kernel(node_features, edge_features, from_idx, to_idx, graph_idx,
           Wn, bn, We, be, Wm1, bm1, Wm2, bm2, Wu1, bu1, Wu2, bu2,
           Wg, bg, Wf, bf):
    bn2 = bn.reshape(1, -1)
    be2 = be.reshape(1, -1)
    bm12 = bm1.reshape(1, -1)
    bm22 = bm2.reshape(1, -1)
    bu12 = bu1.reshape(1, -1)
    bu22 = bu2.reshape(1, -1)
    bg2 = bg.reshape(1, -1)
    bf2 = bf.reshape(1, -1)
    A = Wm1[:_D]
    B = Wm1[_D:2 * _D]
    C = Wm1[2 * _D:]
    U1h = Wu1[:_D]
    U1a = Wu1[_D:2 * _D]
    U1c = Wu1[2 * _D:]
    fi = from_idx.astype(jnp.int32)
    ti = to_idx.astype(jnp.int32)
    idxg = jnp.stack([fi.reshape(_NS, _CPW, _KCH),
                      ti.reshape(_NS, _CPW, _KCH)])
    idxs = ti.reshape(_NC, _NS, _SCH, _KCH)
    zeros = jnp.zeros((_N, _D), jnp.float32)

    h = _encode_h(node_features, Wn, bn2)
    for _ in range(_NPROP):
        st = _gather(h, idxg)
        m = _messages(st, edge_features, A, B, C, We, be2, bm12,
                      Wm2, bm22)
        parts = _scatter(m, idxs, zeros)
        h = _update(h, parts, U1h, U1a, U1c, bu12, Wu2, bu22)
    out = _pool(h, Wg, bg2, Wf, bf2)
    return out[:, 0, 0]


# speedup vs baseline: 2.8404x; 1.0003x over previous
import jax
import jax.numpy as jnp
from jax import lax
from jax.experimental import pallas as pl
from jax.experimental.pallas import tpu as pltpu
from jax.experimental.pallas import tpu_sc as plsc

_N = 10000
_E = 320000
_G = 100
_NPAIRS = 50
_D = 128
_DE = 16
_DEH = 64
_HID = 256
_NPROP = 2

_NC = 2
_NS = 16
_NW = _NC * _NS
_EPW = _E // _NW
_KCH = 80
_NCH = _EPW // _KCH
_RPT = 640
_RLAST = _N - 15 * _RPT

_sc_mesh = plsc.VectorSubcoreMesh(core_axis_name="c", subcore_axis_name="s")


_GEPW = _E // _NS
_CPW = _GEPW // _KCH
_CPG = 2
_GRP = _CPG * _KCH
_NGRP = _CPW // _CPG


def _gather_body(h_hbm, idxg_hbm, st_hbm, idx_v, rows_v, gsem, wsem):
    c = lax.axis_index("c")
    s = lax.axis_index("s")
    pltpu.sync_copy(idxg_hbm.at[c, s], idx_v)
    ebase = s * _GEPW

    def group(g, carry):
        b = lax.rem(g, 2)

        @pl.when(g >= 2)
        def _drain():
            pltpu.make_async_copy(
                rows_v.at[b], st_hbm.at[c, pl.ds(0, _GRP)], wsem).wait()

        cps = [
            pltpu.async_copy(h_hbm.at[idx_v.at[g * _CPG + j]],
                             rows_v.at[b, pl.ds(j * _KCH, _KCH)], gsem)
            for j in range(_CPG)
        ]
        for cp in cps:
            cp.wait()
        off = pl.multiple_of(ebase + g * _GRP, 8)
        pltpu.async_copy(rows_v.at[b], st_hbm.at[c, pl.ds(off, _GRP)], wsem)
        return carry

    lax.fori_loop(0, _NGRP, group, 0)
    pltpu.make_async_copy(rows_v.at[0], st_hbm.at[c, pl.ds(0, _GRP)],
                          wsem).wait()
    pltpu.make_async_copy(rows_v.at[1], st_hbm.at[c, pl.ds(0, _GRP)],
                          wsem).wait()


_gather = pl.kernel(
    _gather_body,
    out_type=jax.ShapeDtypeStruct((_NC, _E, _D), jnp.float32),
    mesh=_sc_mesh,
    scratch_types=[
        pltpu.VMEM((_CPW, _KCH), jnp.int32),
        pltpu.VMEM((2, _GRP, _D), jnp.float32),
        pltpu.SemaphoreType.DMA,
        pltpu.SemaphoreType.DMA,
    ],
)


_SCH = _EPW // _KCH
_SNG = _SCH // _CPG


def _scatter_body(m_hbm, idxs_hbm, z_hbm, part_hbm, idx_v, mrows_v, msem,
                  acc_sh):
    c = lax.axis_index("c")
    s = lax.axis_index("s")
    r0 = pl.multiple_of(s * _RPT, 8)

    @pl.when(s < _NS - 1)
    def _zero_main():
        pltpu.sync_copy(z_hbm.at[pl.ds(r0, _RPT)], acc_sh.at[pl.ds(r0, _RPT)])

    @pl.when(s == _NS - 1)
    def _zero_last():
        pltpu.sync_copy(z_hbm.at[pl.ds(15 * _RPT, _RLAST)],
                        acc_sh.at[pl.ds(15 * _RPT, _RLAST)])

    pltpu.sync_copy(idxs_hbm.at[c, s], idx_v)
    plsc.subcore_barrier()
    ebase = c * (_E // _NC) + s * _EPW
    pltpu.async_copy(m_hbm.at[pl.ds(ebase, _KCH)], mrows_v.at[0], msem)

    def chunk(g, carry):
        b = lax.rem(g, 2)
        pltpu.make_async_copy(m_hbm.at[pl.ds(0, _KCH)], mrows_v.at[b],
                              msem).wait()

        @pl.when(g < _SCH - 1)
        def _prefetch():
            off = pl.multiple_of(ebase + (g + 1) * _KCH, 8)
            pltpu.async_copy(m_hbm.at[pl.ds(off, _KCH)],
                             mrows_v.at[lax.rem(g + 1, 2)], msem)

        pltpu.sync_copy(mrows_v.at[b], acc_sh.at[idx_v.at[g]], add=True)
        return carry

    lax.fori_loop(0, _SCH, chunk, 0)
    plsc.subcore_barrier()

    @pl.when(s < _NS - 1)
    def _dump_main():
        pltpu.sync_copy(acc_sh.at[pl.ds(r0, _RPT)],
                        part_hbm.at[c, pl.ds(r0, _RPT)])

    @pl.when(s == _NS - 1)
    def _dump_last():
        pltpu.sync_copy(acc_sh.at[pl.ds(15 * _RPT, _RLAST)],
                        part_hbm.at[c, pl.ds(15 * _RPT, _RLAST)])


_scatter = pl.kernel(
    _scatter_body,
    out_type=jax.ShapeDtypeStruct((_NC, _N, _D), jnp.float32),
    mesh=_sc_mesh,
    scratch_types=[
        pltpu.VMEM((_SCH, _KCH), jnp.int32),
        pltpu.VMEM((2, _KCH, _D), jnp.float32),
        pltpu.SemaphoreType.DMA,
        pltpu.VMEM_SHARED((_N, _D), jnp.float32),
    ],
)


def _enc_body(nf, Wn, bn, out):
    out[...] = jnp.maximum(nf[...] @ Wn[...] + bn[...], 0.0)


def _encode_h(nf, Wn, bn2):
    blk = 1000
    return pl.pallas_call(
        _enc_body,
        grid=(_N // blk,),
        in_specs=[pl.BlockSpec((blk, _D), lambda i: (i, 0)),
                  pl.BlockSpec((_D, _D), lambda i: (0, 0)),
                  pl.BlockSpec((1, _D), lambda i: (0, 0))],
        out_specs=pl.BlockSpec((blk, _D), lambda i: (i, 0)),
        out_shape=jax.ShapeDtypeStruct((_N, _D), jnp.float32),
    )(nf, Wn, bn2)


_DNN = (((1,), (0,)), ((), ()))


def _msg_body(st, ef, A, B, C, We, be, bm1, Wm2, bm2, out):
    e = jnp.maximum(ef[...] @ We[...] + be[...], 0.0)
    stv = st[...].astype(jnp.bfloat16)
    m1 = (lax.dot_general(stv[0], A[...], _DNN,
                          preferred_element_type=jnp.float32)
          + lax.dot_general(stv[1], B[...], _DNN,
                            preferred_element_type=jnp.float32)
          + e @ C[...] + bm1[...])
    out[...] = lax.dot_general(jnp.maximum(m1, 0.0).astype(jnp.bfloat16),
                               Wm2[...], _DNN,
                               preferred_element_type=jnp.float32) + bm2[...]


def _messages(st, ef, A, B, C, We, be2, bm12, Wm2, bm22):
    blk = 2000
    return pl.pallas_call(
        _msg_body,
        grid=(_E // blk,),
        in_specs=[pl.BlockSpec((_NC, blk, _D), lambda i: (0, i, 0)),
                  pl.BlockSpec((blk, _DE), lambda i: (i, 0)),
                  pl.BlockSpec((_D, _HID), lambda i: (0, 0)),
                  pl.BlockSpec((_D, _HID), lambda i: (0, 0)),
                  pl.BlockSpec((_DEH, _HID), lambda i: (0, 0)),
                  pl.BlockSpec((_DE, _DEH), lambda i: (0, 0)),
                  pl.BlockSpec((1, _DEH), lambda i: (0, 0)),
                  pl.BlockSpec((1, _HID), lambda i: (0, 0)),
                  pl.BlockSpec((_HID, _D), lambda i: (0, 0)),
                  pl.BlockSpec((1, _D), lambda i: (0, 0))],
        out_specs=pl.BlockSpec((blk, _D), lambda i: (i, 0)),
        out_shape=jax.ShapeDtypeStruct((_E, _D), jnp.float32),
    )(st, ef, A, B, C, We, be2, bm12, Wm2, bm22)


def _rsm(x):
    m = jnp.max(x, axis=1, keepdims=True)
    ex = jnp.exp(x - m)
    return ex / jnp.sum(ex, axis=1, keepdims=True)


def _upd_body(hb, pb, U1h, U1a, U1c, bu1, Wu2, bu2, out):
    h = hb[...]
    p = pb[...]
    agg = p[0] + p[1]
    a = h[:_G]
    b = h[_G:]
    dnt = (((1,), (1,)), ((), ()))
    sim = lax.dot_general(a, b, dnt)
    simt = lax.dot_general(b, a, dnt)
    att_a = _rsm(sim) @ b
    att_b = _rsm(simt) @ a
    cross = jnp.concatenate([a - att_a, b - att_b], axis=0)
    u = jnp.maximum(h @ U1h[...] + agg @ U1a[...] + cross @ U1c[...]
                    + bu1[...], 0.0)
    out[...] = u @ Wu2[...] + bu2[...]


def _update(h, parts, U1h, U1a, U1c, bu12, Wu2, bu22):
    blk = 2 * _G
    return pl.pallas_call(
        _upd_body,
        grid=(_NPAIRS,),
        in_specs=[pl.BlockSpec((blk, _D), lambda i: (i, 0)),
                  pl.BlockSpec((_NC, blk, _D), lambda i: (0, i, 0)),
                  pl.BlockSpec((_D, _HID), lambda i: (0, 0)),
                  pl.BlockSpec((_D, _HID), lambda i: (0, 0)),
                  pl.BlockSpec((_D, _HID), lambda i: (0, 0)),
                  pl.BlockSpec((1, _HID), lambda i: (0, 0)),
                  pl.BlockSpec((_HID, _D), lambda i: (0, 0)),
                  pl.BlockSpec((1, _D), lambda i: (0, 0))],
        out_specs=pl.BlockSpec((blk, _D), lambda i: (i, 0)),
        out_shape=jax.ShapeDtypeStruct((_N, _D), jnp.float32),
    )(h, parts, U1h, U1a, U1c, bu12, Wu2, bu22)


def _pool_body(hb, Wg, bg, Wf, bf, out):
    gv = hb[...] @ Wg[...] + bg[...]
    gates = 1.0 / (1.0 + jnp.exp(-gv[:, :_D]))
    gated = gates * gv[:, _D:]
    ga = jnp.sum(gated[:_G], axis=0, keepdims=True)
    gb = jnp.sum(gated[_G:], axis=0, keepdims=True)
    va = ga @ Wf[...] + bf[...]
    vb = gb @ Wf[...] + bf[...]
    r = jnp.maximum(va - vb, 0.0)
    sc = -jnp.sum(r, axis=1, keepdims=True)
    out[...] = jnp.broadcast_to(sc.reshape(1, 1, 1), (1, 1, _D))


def _pool(h, Wg, bg2, Wf, bf2):
    blk = 2 * _G
    return pl.pallas_call(
        _pool_body,
        grid=(_NPAIRS,),
        in_specs=[pl.BlockSpec((blk, _D), lambda i: (i, 0)),
                  pl.BlockSpec((_D, 2 * _D), lambda i: (0, 0)),
                  pl.BlockSpec((1, 2 * _D), lambda i: (0, 0)),
                  pl.BlockSpec((_D, _D), lambda i: (0, 0)),
                  pl.BlockSpec((1, _D), lambda i: (0, 0))],
        out_specs=pl.BlockSpec((1, 1, _D), lambda i: (i, 0, 0)),
        out_shape=jax.ShapeDtypeStruct((_NPAIRS, 1, _D), jnp.float32),
    )(h, Wg, bg2, Wf, bf2)


def kernel(node_features, edge_features, from_idx, to_idx, graph_idx,
           Wn, bn, We, be, Wm1, bm1, Wm2, bm2, Wu1, bu1, Wu2, bu2,
           Wg, bg, Wf, bf):
    bn2 = bn.reshape(1, -1)
    be2 = be.reshape(1, -1)
    bm12 = bm1.reshape(1, -1)
    bm22 = bm2.reshape(1, -1)
    bu12 = bu1.reshape(1, -1)
    bu22 = bu2.reshape(1, -1)
    bg2 = bg.reshape(1, -1)
    bf2 = bf.reshape(1, -1)
    A = Wm1[:_D].astype(jnp.bfloat16)
    B = Wm1[_D:2 * _D].astype(jnp.bfloat16)
    C = Wm1[2 * _D:]
    U1h = Wu1[:_D]
    U1a = Wu1[_D:2 * _D]
    U1c = Wu1[2 * _D:]
    Wm2b = Wm2.astype(jnp.bfloat16)
    fi = from_idx.astype(jnp.int32)
    ti = to_idx.astype(jnp.int32)
    idxg = jnp.stack([fi.reshape(_NS, _CPW, _KCH),
                      ti.reshape(_NS, _CPW, _KCH)])
    idxs = ti.reshape(_NC, _NS, _SCH, _KCH)
    zeros = jnp.zeros((_N, _D), jnp.float32)

    h = _encode_h(node_features, Wn, bn2)
    for _ in range(_NPROP):
        st = _gather(h, idxg)
        m = _messages(st, edge_features, A, B, C, We, be2, bm12,
                      Wm2b, bm22)
        parts = _scatter(m, idxs, zeros)
        h = _update(h, parts, U1h, U1a, U1c, bu12, Wu2, bu22)
    out = _pool(h, Wg, bg2, Wf, bf2)
    return out[:, 0, 0]


# speedup vs baseline: 3.0082x; 1.0591x over previous
import jax
import jax.numpy as jnp
from jax import lax
from jax.experimental import pallas as pl
from jax.experimental.pallas import tpu as pltpu
from jax.experimental.pallas import tpu_sc as plsc

_N = 10000
_E = 320000
_G = 100
_NPAIRS = 50
_D = 128
_DE = 16
_DEH = 64
_HID = 256
_NPROP = 2

_NC = 2
_NS = 16
_NW = _NC * _NS
_EPW = _E // _NW
_KCH = 80
_NCH = _EPW // _KCH
_RPT = 640
_RLAST = _N - 15 * _RPT

_sc_mesh = plsc.VectorSubcoreMesh(core_axis_name="c", subcore_axis_name="s")


_EH = _E // 2
_CPW = _EPW // _KCH
_CPG = 5
_GRP = _CPG * _KCH
_NGRP = _CPW // _CPG


def _gather_body(h_hbm, idxg_hbm, st_hbm, idx_v, rows_v, gsem, wsem):
    c = lax.axis_index("c")
    s = lax.axis_index("s")
    pltpu.sync_copy(idxg_hbm.at[c, s], idx_v)
    ebase = s * _EPW

    def group(g, carry):
        b = lax.rem(g, 2)

        @pl.when(g >= 2)
        def _drain():
            pltpu.make_async_copy(
                rows_v.at[b], st_hbm.at[c, pl.ds(0, _GRP)], wsem).wait()

        cps = [
            pltpu.async_copy(h_hbm.at[idx_v.at[g * _CPG + j]],
                             rows_v.at[b, pl.ds(j * _KCH, _KCH)], gsem)
            for j in range(_CPG)
        ]
        for cp in cps:
            cp.wait()
        off = pl.multiple_of(ebase + g * _GRP, 8)
        pltpu.async_copy(rows_v.at[b], st_hbm.at[c, pl.ds(off, _GRP)], wsem)
        return carry

    lax.fori_loop(0, _NGRP, group, 0)
    pltpu.make_async_copy(rows_v.at[0], st_hbm.at[c, pl.ds(0, _GRP)],
                          wsem).wait()
    pltpu.make_async_copy(rows_v.at[1], st_hbm.at[c, pl.ds(0, _GRP)],
                          wsem).wait()


_gather = pl.kernel(
    _gather_body,
    out_type=jax.ShapeDtypeStruct((_NC, _EH, _D), jnp.float32),
    mesh=_sc_mesh,
    scratch_types=[
        pltpu.VMEM((_CPW, _KCH), jnp.int32),
        pltpu.VMEM((2, _GRP, _D), jnp.float32),
        pltpu.SemaphoreType.DMA,
        pltpu.SemaphoreType.DMA,
    ],
)


_SCH = _EPW // _KCH


def _scatter_pipeline(m_hbm, idx_v, mrows_v, msem, acc_sh, ebase):
    pltpu.async_copy(m_hbm.at[pl.ds(ebase, _KCH)], mrows_v.at[0], msem)

    def chunk(g, carry):
        b = lax.rem(g, 2)
        pltpu.make_async_copy(m_hbm.at[pl.ds(0, _KCH)], mrows_v.at[b],
                              msem).wait()

        @pl.when(g < _SCH - 1)
        def _prefetch():
            off = pl.multiple_of(ebase + (g + 1) * _KCH, 8)
            pltpu.async_copy(m_hbm.at[pl.ds(off, _KCH)],
                             mrows_v.at[lax.rem(g + 1, 2)], msem)

        pltpu.sync_copy(mrows_v.at[b], acc_sh.at[idx_v.at[g]], add=True)
        return carry

    lax.fori_loop(0, _SCH, chunk, 0)


def _scatter_body(m1_hbm, m2_hbm, idxs_hbm, z_hbm, part_hbm, idx_v, mrows_v,
                  msem, acc_sh):
    c = lax.axis_index("c")
    s = lax.axis_index("s")
    r0 = pl.multiple_of(s * _RPT, 8)

    @pl.when(s < _NS - 1)
    def _zero_main():
        pltpu.sync_copy(z_hbm.at[pl.ds(r0, _RPT)], acc_sh.at[pl.ds(r0, _RPT)])

    @pl.when(s == _NS - 1)
    def _zero_last():
        pltpu.sync_copy(z_hbm.at[pl.ds(15 * _RPT, _RLAST)],
                        acc_sh.at[pl.ds(15 * _RPT, _RLAST)])

    pltpu.sync_copy(idxs_hbm.at[c, s], idx_v)
    plsc.subcore_barrier()
    ebase = s * _EPW

    @pl.when(c == 0)
    def _half1():
        _scatter_pipeline(m1_hbm, idx_v, mrows_v, msem, acc_sh, ebase)

    @pl.when(c == 1)
    def _half2():
        _scatter_pipeline(m2_hbm, idx_v, mrows_v, msem, acc_sh, ebase)

    plsc.subcore_barrier()

    @pl.when(s < _NS - 1)
    def _dump_main():
        pltpu.sync_copy(acc_sh.at[pl.ds(r0, _RPT)],
                        part_hbm.at[c, pl.ds(r0, _RPT)])

    @pl.when(s == _NS - 1)
    def _dump_last():
        pltpu.sync_copy(acc_sh.at[pl.ds(15 * _RPT, _RLAST)],
                        part_hbm.at[c, pl.ds(15 * _RPT, _RLAST)])


_scatter = pl.kernel(
    _scatter_body,
    out_type=jax.ShapeDtypeStruct((_NC, _N, _D), jnp.float32),
    mesh=_sc_mesh,
    scratch_types=[
        pltpu.VMEM((_SCH, _KCH), jnp.int32),
        pltpu.VMEM((2, _KCH, _D), jnp.float32),
        pltpu.SemaphoreType.DMA,
        pltpu.VMEM_SHARED((_N, _D), jnp.float32),
    ],
)


def _enc_body(nf, Wn, bn, out):
    out[...] = jnp.maximum(nf[...] @ Wn[...] + bn[...], 0.0)


def _encode_h(nf, Wn, bn2):
    blk = 1000
    return pl.pallas_call(
        _enc_body,
        grid=(_N // blk,),
        in_specs=[pl.BlockSpec((blk, _D), lambda i: (i, 0)),
                  pl.BlockSpec((_D, _D), lambda i: (0, 0)),
                  pl.BlockSpec((1, _D), lambda i: (0, 0))],
        out_specs=pl.BlockSpec((blk, _D), lambda i: (i, 0)),
        out_shape=jax.ShapeDtypeStruct((_N, _D), jnp.float32),
    )(nf, Wn, bn2)


_DNN = (((1,), (0,)), ((), ()))


def _msg_body(st, ef, A, B, C, We, be, bm1, Wm2, bm2, out):
    e = jnp.maximum(ef[...] @ We[...] + be[...], 0.0)
    stv = st[...].astype(jnp.bfloat16)
    m1 = (lax.dot_general(stv[0], A[...], _DNN,
                          preferred_element_type=jnp.float32)
          + lax.dot_general(stv[1], B[...], _DNN,
                            preferred_element_type=jnp.float32)
          + e @ C[...] + bm1[...])
    out[...] = lax.dot_general(jnp.maximum(m1, 0.0).astype(jnp.bfloat16),
                               Wm2[...], _DNN,
                               preferred_element_type=jnp.float32) + bm2[...]


def _messages(st, ef, A, B, C, We, be2, bm12, Wm2, bm22):
    blk = 2000
    return pl.pallas_call(
        _msg_body,
        grid=(_EH // blk,),
        in_specs=[pl.BlockSpec((_NC, blk, _D), lambda i: (0, i, 0)),
                  pl.BlockSpec((blk, _DE), lambda i: (i, 0)),
                  pl.BlockSpec((_D, _HID), lambda i: (0, 0)),
                  pl.BlockSpec((_D, _HID), lambda i: (0, 0)),
                  pl.BlockSpec((_DEH, _HID), lambda i: (0, 0)),
                  pl.BlockSpec((_DE, _DEH), lambda i: (0, 0)),
                  pl.BlockSpec((1, _DEH), lambda i: (0, 0)),
                  pl.BlockSpec((1, _HID), lambda i: (0, 0)),
                  pl.BlockSpec((_HID, _D), lambda i: (0, 0)),
                  pl.BlockSpec((1, _D), lambda i: (0, 0))],
        out_specs=pl.BlockSpec((blk, _D), lambda i: (i, 0)),
        out_shape=jax.ShapeDtypeStruct((_EH, _D), jnp.float32),
    )(st, ef, A, B, C, We, be2, bm12, Wm2, bm22)


def _rsm(x):
    m = jnp.max(x, axis=1, keepdims=True)
    ex = jnp.exp(x - m)
    return ex / jnp.sum(ex, axis=1, keepdims=True)


def _upd_body(hb, pb, U1h, U1a, U1c, bu1, Wu2, bu2, out):
    h = hb[...]
    p = pb[...]
    agg = p[0] + p[1]
    a = h[:_G]
    b = h[_G:]
    dnt = (((1,), (1,)), ((), ()))
    sim = lax.dot_general(a, b, dnt)
    simt = lax.dot_general(b, a, dnt)
    att_a = _rsm(sim) @ b
    att_b = _rsm(simt) @ a
    cross = jnp.concatenate([a - att_a, b - att_b], axis=0)
    u = jnp.maximum(h @ U1h[...] + agg @ U1a[...] + cross @ U1c[...]
                    + bu1[...], 0.0)
    out[...] = u @ Wu2[...] + bu2[...]


def _update(h, parts, U1h, U1a, U1c, bu12, Wu2, bu22):
    blk = 2 * _G
    return pl.pallas_call(
        _upd_body,
        grid=(_NPAIRS,),
        in_specs=[pl.BlockSpec((blk, _D), lambda i: (i, 0)),
                  pl.BlockSpec((_NC, blk, _D), lambda i: (0, i, 0)),
                  pl.BlockSpec((_D, _HID), lambda i: (0, 0)),
                  pl.BlockSpec((_D, _HID), lambda i: (0, 0)),
                  pl.BlockSpec((_D, _HID), lambda i: (0, 0)),
                  pl.BlockSpec((1, _HID), lambda i: (0, 0)),
                  pl.BlockSpec((_HID, _D), lambda i: (0, 0)),
                  pl.BlockSpec((1, _D), lambda i: (0, 0))],
        out_specs=pl.BlockSpec((blk, _D), lambda i: (i, 0)),
        out_shape=jax.ShapeDtypeStruct((_N, _D), jnp.float32),
    )(h, parts, U1h, U1a, U1c, bu12, Wu2, bu22)


def _pool_body(hb, Wg, bg, Wf, bf, out):
    gv = hb[...] @ Wg[...] + bg[...]
    gates = 1.0 / (1.0 + jnp.exp(-gv[:, :_D]))
    gated = gates * gv[:, _D:]
    ga = jnp.sum(gated[:_G], axis=0, keepdims=True)
    gb = jnp.sum(gated[_G:], axis=0, keepdims=True)
    va = ga @ Wf[...] + bf[...]
    vb = gb @ Wf[...] + bf[...]
    r = jnp.maximum(va - vb, 0.0)
    sc = -jnp.sum(r, axis=1, keepdims=True)
    out[...] = jnp.broadcast_to(sc.reshape(1, 1, 1), (1, 1, _D))


def _pool(h, Wg, bg2, Wf, bf2):
    blk = 2 * _G
    return pl.pallas_call(
        _pool_body,
        grid=(_NPAIRS,),
        in_specs=[pl.BlockSpec((blk, _D), lambda i: (i, 0)),
                  pl.BlockSpec((_D, 2 * _D), lambda i: (0, 0)),
                  pl.BlockSpec((1, 2 * _D), lambda i: (0, 0)),
                  pl.BlockSpec((_D, _D), lambda i: (0, 0)),
                  pl.BlockSpec((1, _D), lambda i: (0, 0))],
        out_specs=pl.BlockSpec((1, 1, _D), lambda i: (i, 0, 0)),
        out_shape=jax.ShapeDtypeStruct((_NPAIRS, 1, _D), jnp.float32),
    )(h, Wg, bg2, Wf, bf2)


def kernel(node_features, edge_features, from_idx, to_idx, graph_idx,
           Wn, bn, We, be, Wm1, bm1, Wm2, bm2, Wu1, bu1, Wu2, bu2,
           Wg, bg, Wf, bf):
    bn2 = bn.reshape(1, -1)
    be2 = be.reshape(1, -1)
    bm12 = bm1.reshape(1, -1)
    bm22 = bm2.reshape(1, -1)
    bu12 = bu1.reshape(1, -1)
    bu22 = bu2.reshape(1, -1)
    bg2 = bg.reshape(1, -1)
    bf2 = bf.reshape(1, -1)
    A = Wm1[:_D].astype(jnp.bfloat16)
    B = Wm1[_D:2 * _D].astype(jnp.bfloat16)
    C = Wm1[2 * _D:]
    U1h = Wu1[:_D]
    U1a = Wu1[_D:2 * _D]
    U1c = Wu1[2 * _D:]
    Wm2b = Wm2.astype(jnp.bfloat16)
    fi = from_idx.astype(jnp.int32)
    ti = to_idx.astype(jnp.int32)
    idxg1 = jnp.stack([fi[:_EH].reshape(_NS, _CPW, _KCH),
                       ti[:_EH].reshape(_NS, _CPW, _KCH)])
    idxg2 = jnp.stack([fi[_EH:].reshape(_NS, _CPW, _KCH),
                       ti[_EH:].reshape(_NS, _CPW, _KCH)])
    idxs = ti.reshape(_NC, _NS, _SCH, _KCH)
    ef1 = edge_features[:_EH]
    ef2 = edge_features[_EH:]
    zeros = jnp.zeros((_N, _D), jnp.float32)

    h = _encode_h(node_features, Wn, bn2)
    for _ in range(_NPROP):
        st1 = _gather(h, idxg1)
        m1 = _messages(st1, ef1, A, B, C, We, be2, bm12, Wm2b, bm22)
        st2 = _gather(h, idxg2)
        m2 = _messages(st2, ef2, A, B, C, We, be2, bm12, Wm2b, bm22)
        parts = _scatter(m1, m2, idxs, zeros)
        h = _update(h, parts, U1h, U1a, U1c, bu12, Wu2, bu22)
    out = _pool(h, Wg, bg2, Wf, bf2)
    return out[:, 0, 0]


# speedup vs baseline: 3.0683x; 1.0200x over previous
import jax
import jax.numpy as jnp
from jax import lax
from jax.experimental import pallas as pl
from jax.experimental.pallas import tpu as pltpu
from jax.experimental.pallas import tpu_sc as plsc

_N = 10000
_E = 320000
_G = 100
_NPAIRS = 50
_D = 128
_DE = 16
_DEH = 64
_HID = 256
_NPROP = 2

_NC = 2
_NS = 16
_NW = _NC * _NS
_EPW = _E // _NW
_KCH = 80
_NCH = _EPW // _KCH
_RPT = 640
_RLAST = _N - 15 * _RPT

_sc_mesh = plsc.VectorSubcoreMesh(core_axis_name="c", subcore_axis_name="s")


_EH = _E // 2
_CPW = _EPW // _KCH
_CPG = 5
_GRP = _CPG * _KCH
_NGRP = _CPW // _CPG


def _gather_body(h_hbm, idxg_hbm, st_hbm, idx_v, rows_v, gsem, wsem):
    c = lax.axis_index("c")
    s = lax.axis_index("s")
    pltpu.sync_copy(idxg_hbm.at[c, s], idx_v)
    ebase = s * _EPW

    def group(g, carry):
        b = lax.rem(g, 2)

        @pl.when(g >= 2)
        def _drain():
            pltpu.make_async_copy(
                rows_v.at[b], st_hbm.at[c, pl.ds(0, _GRP)], wsem).wait()

        cps = [
            pltpu.async_copy(h_hbm.at[idx_v.at[g * _CPG + j]],
                             rows_v.at[b, pl.ds(j * _KCH, _KCH)], gsem)
            for j in range(_CPG)
        ]
        for cp in cps:
            cp.wait()
        off = pl.multiple_of(ebase + g * _GRP, 8)
        pltpu.async_copy(rows_v.at[b], st_hbm.at[c, pl.ds(off, _GRP)], wsem)
        return carry

    lax.fori_loop(0, _NGRP, group, 0)
    pltpu.make_async_copy(rows_v.at[0], st_hbm.at[c, pl.ds(0, _GRP)],
                          wsem).wait()
    pltpu.make_async_copy(rows_v.at[1], st_hbm.at[c, pl.ds(0, _GRP)],
                          wsem).wait()


_gather = pl.kernel(
    _gather_body,
    out_type=jax.ShapeDtypeStruct((_NC, _EH, _D), jnp.float32),
    mesh=_sc_mesh,
    scratch_types=[
        pltpu.VMEM((_CPW, _KCH), jnp.int32),
        pltpu.VMEM((2, _GRP, _D), jnp.float32),
        pltpu.SemaphoreType.DMA,
        pltpu.SemaphoreType.DMA,
    ],
)


_K2 = 40
_SPW = _EH // _NW
_SC2 = _SPW // _K2


def _scatter_body(m_hbm, idxs_hbm, init_hbm, part_hbm, idx_v, mrows_v,
                  msem, acc_sh):
    c = lax.axis_index("c")
    s = lax.axis_index("s")
    r0 = pl.multiple_of(s * _RPT, 8)

    @pl.when(s < _NS - 1)
    def _init_main():
        pltpu.sync_copy(init_hbm.at[c, pl.ds(r0, _RPT)],
                        acc_sh.at[pl.ds(r0, _RPT)])

    @pl.when(s == _NS - 1)
    def _init_last():
        pltpu.sync_copy(init_hbm.at[c, pl.ds(15 * _RPT, _RLAST)],
                        acc_sh.at[pl.ds(15 * _RPT, _RLAST)])

    pltpu.sync_copy(idxs_hbm.at[c, s], idx_v)
    plsc.subcore_barrier()
    ebase = (c * _NS + s) * _SPW
    pltpu.async_copy(m_hbm.at[pl.ds(ebase, _K2)], mrows_v.at[0], msem)

    def chunk(g, carry):
        b = lax.rem(g, 2)
        pltpu.make_async_copy(m_hbm.at[pl.ds(0, _K2)], mrows_v.at[b],
                              msem).wait()

        @pl.when(g < _SC2 - 1)
        def _prefetch():
            off = pl.multiple_of(ebase + (g + 1) * _K2, 8)
            pltpu.async_copy(m_hbm.at[pl.ds(off, _K2)],
                             mrows_v.at[lax.rem(g + 1, 2)], msem)

        pltpu.sync_copy(mrows_v.at[b], acc_sh.at[idx_v.at[g]], add=True)
        return carry

    lax.fori_loop(0, _SC2, chunk, 0)
    plsc.subcore_barrier()

    @pl.when(s < _NS - 1)
    def _dump_main():
        pltpu.sync_copy(acc_sh.at[pl.ds(r0, _RPT)],
                        part_hbm.at[c, pl.ds(r0, _RPT)])

    @pl.when(s == _NS - 1)
    def _dump_last():
        pltpu.sync_copy(acc_sh.at[pl.ds(15 * _RPT, _RLAST)],
                        part_hbm.at[c, pl.ds(15 * _RPT, _RLAST)])


_scatter = pl.kernel(
    _scatter_body,
    out_type=jax.ShapeDtypeStruct((_NC, _N, _D), jnp.float32),
    mesh=_sc_mesh,
    scratch_types=[
        pltpu.VMEM((_SC2, _K2), jnp.int32),
        pltpu.VMEM((2, _K2, _D), jnp.float32),
        pltpu.SemaphoreType.DMA,
        pltpu.VMEM_SHARED((_N, _D), jnp.float32),
    ],
)


def _enc_body(nf, Wn, bn, out):
    out[...] = jnp.maximum(nf[...] @ Wn[...] + bn[...], 0.0)


def _encode_h(nf, Wn, bn2):
    blk = 1000
    return pl.pallas_call(
        _enc_body,
        grid=(_N // blk,),
        in_specs=[pl.BlockSpec((blk, _D), lambda i: (i, 0)),
                  pl.BlockSpec((_D, _D), lambda i: (0, 0)),
                  pl.BlockSpec((1, _D), lambda i: (0, 0))],
        out_specs=pl.BlockSpec((blk, _D), lambda i: (i, 0)),
        out_shape=jax.ShapeDtypeStruct((_N, _D), jnp.float32),
    )(nf, Wn, bn2)


_DNN = (((1,), (0,)), ((), ()))


def _msg_body(st, ef, A, B, C, We, be, bm1, Wm2, bm2, out):
    e = jnp.maximum(ef[...] @ We[...] + be[...], 0.0)
    stv = st[...].astype(jnp.bfloat16)
    m1 = (lax.dot_general(stv[0], A[...], _DNN,
                          preferred_element_type=jnp.float32)
          + lax.dot_general(stv[1], B[...], _DNN,
                            preferred_element_type=jnp.float32)
          + e @ C[...] + bm1[...])
    out[...] = lax.dot_general(jnp.maximum(m1, 0.0).astype(jnp.bfloat16),
                               Wm2[...], _DNN,
                               preferred_element_type=jnp.float32) + bm2[...]


def _messages(st, ef, A, B, C, We, be2, bm12, Wm2, bm22):
    blk = 2000
    return pl.pallas_call(
        _msg_body,
        grid=(_EH // blk,),
        in_specs=[pl.BlockSpec((_NC, blk, _D), lambda i: (0, i, 0)),
                  pl.BlockSpec((blk, _DE), lambda i: (i, 0)),
                  pl.BlockSpec((_D, _HID), lambda i: (0, 0)),
                  pl.BlockSpec((_D, _HID), lambda i: (0, 0)),
                  pl.BlockSpec((_DEH, _HID), lambda i: (0, 0)),
                  pl.BlockSpec((_DE, _DEH), lambda i: (0, 0)),
                  pl.BlockSpec((1, _DEH), lambda i: (0, 0)),
                  pl.BlockSpec((1, _HID), lambda i: (0, 0)),
                  pl.BlockSpec((_HID, _D), lambda i: (0, 0)),
                  pl.BlockSpec((1, _D), lambda i: (0, 0))],
        out_specs=pl.BlockSpec((blk, _D), lambda i: (i, 0)),
        out_shape=jax.ShapeDtypeStruct((_EH, _D), jnp.float32),
    )(st, ef, A, B, C, We, be2, bm12, Wm2, bm22)


def _rsm(x):
    m = jnp.max(x, axis=1, keepdims=True)
    ex = jnp.exp(x - m)
    return ex / jnp.sum(ex, axis=1, keepdims=True)


def _upd_body(hb, pb, U1h, U1a, U1c, bu1, Wu2, bu2, out):
    h = hb[...]
    p = pb[...]
    agg = p[0] + p[1]
    a = h[:_G]
    b = h[_G:]
    dnt = (((1,), (1,)), ((), ()))
    sim = lax.dot_general(a, b, dnt)
    simt = lax.dot_general(b, a, dnt)
    att_a = _rsm(sim) @ b
    att_b = _rsm(simt) @ a
    cross = jnp.concatenate([a - att_a, b - att_b], axis=0)
    u = jnp.maximum(h @ U1h[...] + agg @ U1a[...] + cross @ U1c[...]
                    + bu1[...], 0.0)
    out[...] = u @ Wu2[...] + bu2[...]


def _update(h, parts, U1h, U1a, U1c, bu12, Wu2, bu22):
    blk = 2 * _G
    return pl.pallas_call(
        _upd_body,
        grid=(_NPAIRS,),
        in_specs=[pl.BlockSpec((blk, _D), lambda i: (i, 0)),
                  pl.BlockSpec((_NC, blk, _D), lambda i: (0, i, 0)),
                  pl.BlockSpec((_D, _HID), lambda i: (0, 0)),
                  pl.BlockSpec((_D, _HID), lambda i: (0, 0)),
                  pl.BlockSpec((_D, _HID), lambda i: (0, 0)),
                  pl.BlockSpec((1, _HID), lambda i: (0, 0)),
                  pl.BlockSpec((_HID, _D), lambda i: (0, 0)),
                  pl.BlockSpec((1, _D), lambda i: (0, 0))],
        out_specs=pl.BlockSpec((blk, _D), lambda i: (i, 0)),
        out_shape=jax.ShapeDtypeStruct((_N, _D), jnp.float32),
    )(h, parts, U1h, U1a, U1c, bu12, Wu2, bu22)


def _pool_body(hb, Wg, bg, Wf, bf, out):
    gv = hb[...] @ Wg[...] + bg[...]
    gates = 1.0 / (1.0 + jnp.exp(-gv[:, :_D]))
    gated = gates * gv[:, _D:]
    ga = jnp.sum(gated[:_G], axis=0, keepdims=True)
    gb = jnp.sum(gated[_G:], axis=0, keepdims=True)
    va = ga @ Wf[...] + bf[...]
    vb = gb @ Wf[...] + bf[...]
    r = jnp.maximum(va - vb, 0.0)
    sc = -jnp.sum(r, axis=1, keepdims=True)
    out[...] = jnp.broadcast_to(sc.reshape(1, 1, 1), (1, 1, _D))


def _pool(h, Wg, bg2, Wf, bf2):
    blk = 2 * _G
    return pl.pallas_call(
        _pool_body,
        grid=(_NPAIRS,),
        in_specs=[pl.BlockSpec((blk, _D), lambda i: (i, 0)),
                  pl.BlockSpec((_D, 2 * _D), lambda i: (0, 0)),
                  pl.BlockSpec((1, 2 * _D), lambda i: (0, 0)),
                  pl.BlockSpec((_D, _D), lambda i: (0, 0)),
                  pl.BlockSpec((1, _D), lambda i: (0, 0))],
        out_specs=pl.BlockSpec((1, 1, _D), lambda i: (i, 0, 0)),
        out_shape=jax.ShapeDtypeStruct((_NPAIRS, 1, _D), jnp.float32),
    )(h, Wg, bg2, Wf, bf2)


def kernel(node_features, edge_features, from_idx, to_idx, graph_idx,
           Wn, bn, We, be, Wm1, bm1, Wm2, bm2, Wu1, bu1, Wu2, bu2,
           Wg, bg, Wf, bf):
    bn2 = bn.reshape(1, -1)
    be2 = be.reshape(1, -1)
    bm12 = bm1.reshape(1, -1)
    bm22 = bm2.reshape(1, -1)
    bu12 = bu1.reshape(1, -1)
    bu22 = bu2.reshape(1, -1)
    bg2 = bg.reshape(1, -1)
    bf2 = bf.reshape(1, -1)
    A = Wm1[:_D].astype(jnp.bfloat16)
    B = Wm1[_D:2 * _D].astype(jnp.bfloat16)
    C = Wm1[2 * _D:]
    U1h = Wu1[:_D]
    U1a = Wu1[_D:2 * _D]
    U1c = Wu1[2 * _D:]
    Wm2b = Wm2.astype(jnp.bfloat16)
    fi = from_idx.astype(jnp.int32)
    ti = to_idx.astype(jnp.int32)
    idxg1 = jnp.stack([fi[:_EH].reshape(_NS, _CPW, _KCH),
                       ti[:_EH].reshape(_NS, _CPW, _KCH)])
    idxg2 = jnp.stack([fi[_EH:].reshape(_NS, _CPW, _KCH),
                       ti[_EH:].reshape(_NS, _CPW, _KCH)])
    idxs1 = ti[:_EH].reshape(_NC, _NS, _SC2, _K2)
    idxs2 = ti[_EH:].reshape(_NC, _NS, _SC2, _K2)
    ef1 = edge_features[:_EH]
    ef2 = edge_features[_EH:]
    zeros = jnp.zeros((_NC, _N, _D), jnp.float32)

    h = _encode_h(node_features, Wn, bn2)
    for _ in range(_NPROP):
        st1 = _gather(h, idxg1)
        m1 = _messages(st1, ef1, A, B, C, We, be2, bm12, Wm2b, bm22)
        st2 = _gather(h, idxg2)
        m2 = _messages(st2, ef2, A, B, C, We, be2, bm12, Wm2b, bm22)
        parts1 = _scatter(m1, idxs1, zeros)
        parts = _scatter(m2, idxs2, parts1)
        h = _update(h, parts, U1h, U1a, U1c, bu12, Wu2, bu22)
    out = _pool(h, Wg, bg2, Wf, bf2)
    return out[:, 0, 0]


# speedup vs baseline: 3.2024x; 1.0437x over previous
import jax
import jax.numpy as jnp
from jax import lax
from jax.experimental import pallas as pl
from jax.experimental.pallas import tpu as pltpu
from jax.experimental.pallas import tpu_sc as plsc

_N = 10000
_E = 320000
_G = 100
_NPAIRS = 50
_D = 128
_DE = 16
_DEH = 64
_HID = 256
_NPROP = 2

_NC = 2
_NS = 16
_NW = _NC * _NS
_EPW = _E // _NW
_KCH = 80
_NCH = _EPW // _KCH
_RPT = 640
_RLAST = _N - 15 * _RPT

_sc_mesh = plsc.VectorSubcoreMesh(core_axis_name="c", subcore_axis_name="s")


_EH = _E // 2
_CPW = _EPW // _KCH
_CPG = 5
_GRP = _CPG * _KCH
_NGRP = _CPW // _CPG


def _gather_body(h_hbm, idxg_hbm, st_hbm, idx_v, rows_v, gsem, wsem):
    c = lax.axis_index("c")
    s = lax.axis_index("s")
    pltpu.sync_copy(idxg_hbm.at[c, s], idx_v)
    ebase = s * _EPW

    def group(g, carry):
        b = lax.rem(g, 2)

        @pl.when(g >= 2)
        def _drain():
            pltpu.make_async_copy(
                rows_v.at[b], st_hbm.at[c, pl.ds(0, _GRP)], wsem).wait()

        cps = [
            pltpu.async_copy(h_hbm.at[idx_v.at[g * _CPG + j]],
                             rows_v.at[b, pl.ds(j * _KCH, _KCH)], gsem)
            for j in range(_CPG)
        ]
        for cp in cps:
            cp.wait()
        off = pl.multiple_of(ebase + g * _GRP, 8)
        pltpu.async_copy(rows_v.at[b], st_hbm.at[c, pl.ds(off, _GRP)], wsem)
        return carry

    lax.fori_loop(0, _NGRP, group, 0)
    pltpu.make_async_copy(rows_v.at[0], st_hbm.at[c, pl.ds(0, _GRP)],
                          wsem).wait()
    pltpu.make_async_copy(rows_v.at[1], st_hbm.at[c, pl.ds(0, _GRP)],
                          wsem).wait()


_gather = pl.kernel(
    _gather_body,
    out_type=jax.ShapeDtypeStruct((_NC, _EH, _D), jnp.float32),
    mesh=_sc_mesh,
    scratch_types=[
        pltpu.VMEM((_CPW, _KCH), jnp.int32),
        pltpu.VMEM((2, _GRP, _D), jnp.float32),
        pltpu.SemaphoreType.DMA,
        pltpu.SemaphoreType.DMA,
    ],
)


_K2 = 40
_SPW = _EH // _NW
_SC2 = _SPW // _K2


def _scatter_body(m_hbm, idxs_hbm, init_hbm, part_hbm, idx_v, mrows_v,
                  msem, acc_sh):
    c = lax.axis_index("c")
    s = lax.axis_index("s")
    r0 = pl.multiple_of(s * _RPT, 8)

    @pl.when(s < _NS - 1)
    def _init_main():
        pltpu.sync_copy(init_hbm.at[c, pl.ds(r0, _RPT)],
                        acc_sh.at[pl.ds(r0, _RPT)])

    @pl.when(s == _NS - 1)
    def _init_last():
        pltpu.sync_copy(init_hbm.at[c, pl.ds(15 * _RPT, _RLAST)],
                        acc_sh.at[pl.ds(15 * _RPT, _RLAST)])

    pltpu.sync_copy(idxs_hbm.at[c, s], idx_v)
    plsc.subcore_barrier()
    ebase = (c * _NS + s) * _SPW
    pltpu.async_copy(m_hbm.at[pl.ds(ebase, _K2)], mrows_v.at[0], msem)

    def chunk(g, carry):
        b = lax.rem(g, 2)
        pltpu.make_async_copy(m_hbm.at[pl.ds(0, _K2)], mrows_v.at[b],
                              msem).wait()

        @pl.when(g < _SC2 - 1)
        def _prefetch():
            off = pl.multiple_of(ebase + (g + 1) * _K2, 8)
            pltpu.async_copy(m_hbm.at[pl.ds(off, _K2)],
                             mrows_v.at[lax.rem(g + 1, 2)], msem)

        pltpu.sync_copy(mrows_v.at[b], acc_sh.at[idx_v.at[g]], add=True)
        return carry

    lax.fori_loop(0, _SC2, chunk, 0)
    plsc.subcore_barrier()

    @pl.when(s < _NS - 1)
    def _dump_main():
        pltpu.sync_copy(acc_sh.at[pl.ds(r0, _RPT)],
                        part_hbm.at[c, pl.ds(r0, _RPT)])

    @pl.when(s == _NS - 1)
    def _dump_last():
        pltpu.sync_copy(acc_sh.at[pl.ds(15 * _RPT, _RLAST)],
                        part_hbm.at[c, pl.ds(15 * _RPT, _RLAST)])


_scatter = pl.kernel(
    _scatter_body,
    out_type=jax.ShapeDtypeStruct((_NC, _N, _D), jnp.float32),
    mesh=_sc_mesh,
    scratch_types=[
        pltpu.VMEM((_SC2, _K2), jnp.int32),
        pltpu.VMEM((2, _K2, _D), jnp.float32),
        pltpu.SemaphoreType.DMA,
        pltpu.VMEM_SHARED((_N, _D), jnp.float32),
    ],
)


def _enc_body(nf, Wn, bn, out):
    out[...] = jnp.maximum(nf[...] @ Wn[...] + bn[...], 0.0)


def _encode_h(nf, Wn, bn2):
    blk = 1000
    return pl.pallas_call(
        _enc_body,
        grid=(_N // blk,),
        in_specs=[pl.BlockSpec((blk, _D), lambda i: (i, 0)),
                  pl.BlockSpec((_D, _D), lambda i: (0, 0)),
                  pl.BlockSpec((1, _D), lambda i: (0, 0))],
        out_specs=pl.BlockSpec((blk, _D), lambda i: (i, 0)),
        out_shape=jax.ShapeDtypeStruct((_N, _D), jnp.float32),
    )(nf, Wn, bn2)


_DNN = (((1,), (0,)), ((), ()))


def _msg_body(st, ef, A, B, C, We, be, bm1, Wm2, bm2, out):
    e = jnp.maximum(ef[...] @ We[...] + be[...], 0.0)
    stv = st[...].astype(jnp.bfloat16)
    m1 = (lax.dot_general(stv[0], A[...], _DNN,
                          preferred_element_type=jnp.float32)
          + lax.dot_general(stv[1], B[...], _DNN,
                            preferred_element_type=jnp.float32)
          + e @ C[...] + bm1[...])
    out[...] = lax.dot_general(jnp.maximum(m1, 0.0).astype(jnp.bfloat16),
                               Wm2[...], _DNN,
                               preferred_element_type=jnp.float32) + bm2[...]


def _messages(st, ef, A, B, C, We, be2, bm12, Wm2, bm22):
    blk = 2000
    return pl.pallas_call(
        _msg_body,
        grid=(_EH // blk,),
        in_specs=[pl.BlockSpec((_NC, blk, _D), lambda i: (0, i, 0)),
                  pl.BlockSpec((blk, _DE), lambda i: (i, 0)),
                  pl.BlockSpec((_D, _HID), lambda i: (0, 0)),
                  pl.BlockSpec((_D, _HID), lambda i: (0, 0)),
                  pl.BlockSpec((_DEH, _HID), lambda i: (0, 0)),
                  pl.BlockSpec((_DE, _DEH), lambda i: (0, 0)),
                  pl.BlockSpec((1, _DEH), lambda i: (0, 0)),
                  pl.BlockSpec((1, _HID), lambda i: (0, 0)),
                  pl.BlockSpec((_HID, _D), lambda i: (0, 0)),
                  pl.BlockSpec((1, _D), lambda i: (0, 0))],
        out_specs=pl.BlockSpec((blk, _D), lambda i: (i, 0)),
        out_shape=jax.ShapeDtypeStruct((_EH, _D), jnp.float32),
    )(st, ef, A, B, C, We, be2, bm12, Wm2, bm22)


def _rsm(x):
    m = jnp.max(x, axis=1, keepdims=True)
    ex = jnp.exp(x - m)
    return ex / jnp.sum(ex, axis=1, keepdims=True)


_PPB = 5


def _upd_body(hb, pb, U1h, U1a, U1c, bu1, Wu2, bu2, out):
    h = hb[...]
    p = pb[...]
    agg = p[0] + p[1]
    dnt = (((1,), (1,)), ((), ()))
    crosses = []
    for q in range(_PPB):
        a = h[2 * _G * q:2 * _G * q + _G]
        b = h[2 * _G * q + _G:2 * _G * (q + 1)]
        sim = lax.dot_general(a, b, dnt)
        simt = lax.dot_general(b, a, dnt)
        att_a = _rsm(sim) @ b
        att_b = _rsm(simt) @ a
        crosses.append(a - att_a)
        crosses.append(b - att_b)
    cross = jnp.concatenate(crosses, axis=0)
    u = jnp.maximum(h @ U1h[...] + agg @ U1a[...] + cross @ U1c[...]
                    + bu1[...], 0.0)
    out[...] = u @ Wu2[...] + bu2[...]


def _update(h, parts, U1h, U1a, U1c, bu12, Wu2, bu22):
    blk = 2 * _G * _PPB
    return pl.pallas_call(
        _upd_body,
        grid=(_NPAIRS // _PPB,),
        in_specs=[pl.BlockSpec((blk, _D), lambda i: (i, 0)),
                  pl.BlockSpec((_NC, blk, _D), lambda i: (0, i, 0)),
                  pl.BlockSpec((_D, _HID), lambda i: (0, 0)),
                  pl.BlockSpec((_D, _HID), lambda i: (0, 0)),
                  pl.BlockSpec((_D, _HID), lambda i: (0, 0)),
                  pl.BlockSpec((1, _HID), lambda i: (0, 0)),
                  pl.BlockSpec((_HID, _D), lambda i: (0, 0)),
                  pl.BlockSpec((1, _D), lambda i: (0, 0))],
        out_specs=pl.BlockSpec((blk, _D), lambda i: (i, 0)),
        out_shape=jax.ShapeDtypeStruct((_N, _D), jnp.float32),
    )(h, parts, U1h, U1a, U1c, bu12, Wu2, bu22)


def _pool_body(hb, Wg, bg, Wf, bf, out):
    gv = hb[...] @ Wg[...] + bg[...]
    gates = 1.0 / (1.0 + jnp.exp(-gv[:, :_D]))
    gated = gates * gv[:, _D:]
    sums = []
    for q in range(_PPB):
        sums.append(jnp.sum(gated[2 * _G * q:2 * _G * q + _G], axis=0,
                            keepdims=True))
    for q in range(_PPB):
        sums.append(jnp.sum(gated[2 * _G * q + _G:2 * _G * (q + 1)], axis=0,
                            keepdims=True))
    gs = jnp.concatenate(sums, axis=0)
    gvec = gs @ Wf[...] + bf[...]
    va = gvec[:_PPB]
    vb = gvec[_PPB:]
    r = jnp.maximum(va - vb, 0.0)
    sc = -jnp.sum(r, axis=1, keepdims=True)
    out[...] = jnp.broadcast_to(sc.reshape(_PPB, 1, 1), (_PPB, 1, _D))


def _pool(h, Wg, bg2, Wf, bf2):
    blk = 2 * _G * _PPB
    return pl.pallas_call(
        _pool_body,
        grid=(_NPAIRS // _PPB,),
        in_specs=[pl.BlockSpec((blk, _D), lambda i: (i, 0)),
                  pl.BlockSpec((_D, 2 * _D), lambda i: (0, 0)),
                  pl.BlockSpec((1, 2 * _D), lambda i: (0, 0)),
                  pl.BlockSpec((_D, _D), lambda i: (0, 0)),
                  pl.BlockSpec((1, _D), lambda i: (0, 0))],
        out_specs=pl.BlockSpec((_PPB, 1, _D), lambda i: (i, 0, 0)),
        out_shape=jax.ShapeDtypeStruct((_NPAIRS, 1, _D), jnp.float32),
    )(h, Wg, bg2, Wf, bf2)


def kernel(node_features, edge_features, from_idx, to_idx, graph_idx,
           Wn, bn, We, be, Wm1, bm1, Wm2, bm2, Wu1, bu1, Wu2, bu2,
           Wg, bg, Wf, bf):
    bn2 = bn.reshape(1, -1)
    be2 = be.reshape(1, -1)
    bm12 = bm1.reshape(1, -1)
    bm22 = bm2.reshape(1, -1)
    bu12 = bu1.reshape(1, -1)
    bu22 = bu2.reshape(1, -1)
    bg2 = bg.reshape(1, -1)
    bf2 = bf.reshape(1, -1)
    A = Wm1[:_D].astype(jnp.bfloat16)
    B = Wm1[_D:2 * _D].astype(jnp.bfloat16)
    C = Wm1[2 * _D:]
    U1h = Wu1[:_D]
    U1a = Wu1[_D:2 * _D]
    U1c = Wu1[2 * _D:]
    Wm2b = Wm2.astype(jnp.bfloat16)
    fi = from_idx.astype(jnp.int32)
    ti = to_idx.astype(jnp.int32)
    idxg1 = jnp.stack([fi[:_EH].reshape(_NS, _CPW, _KCH),
                       ti[:_EH].reshape(_NS, _CPW, _KCH)])
    idxg2 = jnp.stack([fi[_EH:].reshape(_NS, _CPW, _KCH),
                       ti[_EH:].reshape(_NS, _CPW, _KCH)])
    idxs1 = ti[:_EH].reshape(_NC, _NS, _SC2, _K2)
    idxs2 = ti[_EH:].reshape(_NC, _NS, _SC2, _K2)
    ef1 = edge_features[:_EH]
    ef2 = edge_features[_EH:]
    zeros = jnp.zeros((_NC, _N, _D), jnp.float32)

    h = _encode_h(node_features, Wn, bn2)
    for _ in range(_NPROP):
        st1 = _gather(h, idxg1)
        m1 = _messages(st1, ef1, A, B, C, We, be2, bm12, Wm2b, bm22)
        st2 = _gather(h, idxg2)
        m2 = _messages(st2, ef2, A, B, C, We, be2, bm12, Wm2b, bm22)
        parts1 = _scatter(m1, idxs1, zeros)
        parts = _scatter(m2, idxs2, parts1)
        h = _update(h, parts, U1h, U1a, U1c, bu12, Wu2, bu22)
    out = _pool(h, Wg, bg2, Wf, bf2)
    return out[:, 0, 0]


# speedup vs baseline: 3.3073x; 1.0327x over previous
import jax
import jax.numpy as jnp
from jax import lax
from jax.experimental import pallas as pl
from jax.experimental.pallas import tpu as pltpu
from jax.experimental.pallas import tpu_sc as plsc

_N = 10000
_E = 320000
_G = 100
_NPAIRS = 50
_D = 128
_DE = 16
_DEH = 64
_HID = 256
_NPROP = 2

_NC = 2
_NS = 16
_NW = _NC * _NS
_EPW = _E // _NW
_KCH = 80
_NCH = _EPW // _KCH
_RPT = 640
_RLAST = _N - 15 * _RPT

_sc_mesh = plsc.VectorSubcoreMesh(core_axis_name="c", subcore_axis_name="s")


_EH = _E // 2
_CPW = _EPW // _KCH
_CPG = 5
_GRP = _CPG * _KCH
_NGRP = _CPW // _CPG


def _gather_body(h_hbm, idxg_hbm, st_hbm, idx_v, rows_v, gsem, wsem):
    c = lax.axis_index("c")
    s = lax.axis_index("s")
    pltpu.sync_copy(idxg_hbm.at[c, s], idx_v)
    ebase = s * _EPW

    def group(g, carry):
        b = lax.rem(g, 2)

        @pl.when(g >= 2)
        def _drain():
            pltpu.make_async_copy(
                rows_v.at[b], st_hbm.at[c, pl.ds(0, _GRP)], wsem).wait()

        cps = [
            pltpu.async_copy(h_hbm.at[idx_v.at[g * _CPG + j]],
                             rows_v.at[b, pl.ds(j * _KCH, _KCH)], gsem)
            for j in range(_CPG)
        ]
        for cp in cps:
            cp.wait()
        off = pl.multiple_of(ebase + g * _GRP, 8)
        pltpu.async_copy(rows_v.at[b], st_hbm.at[c, pl.ds(off, _GRP)], wsem)
        return carry

    lax.fori_loop(0, _NGRP, group, 0)
    pltpu.make_async_copy(rows_v.at[0], st_hbm.at[c, pl.ds(0, _GRP)],
                          wsem).wait()
    pltpu.make_async_copy(rows_v.at[1], st_hbm.at[c, pl.ds(0, _GRP)],
                          wsem).wait()


_gather = pl.kernel(
    _gather_body,
    out_type=jax.ShapeDtypeStruct((_NC, _EH, _D), jnp.float32),
    mesh=_sc_mesh,
    scratch_types=[
        pltpu.VMEM((_CPW, _KCH), jnp.int32),
        pltpu.VMEM((2, _GRP, _D), jnp.float32),
        pltpu.SemaphoreType.DMA,
        pltpu.SemaphoreType.DMA,
    ],
)


_K2 = 40
_SPW = _EH // _NW
_SC2 = _SPW // _K2


def _scatter_body(m_hbm, idxs_hbm, init_hbm, part_hbm, idx_v, mrows_v,
                  msem, acc_sh):
    c = lax.axis_index("c")
    s = lax.axis_index("s")
    r0 = pl.multiple_of(s * _RPT, 8)

    @pl.when(s < _NS - 1)
    def _init_main():
        pltpu.sync_copy(init_hbm.at[c, pl.ds(r0, _RPT)],
                        acc_sh.at[pl.ds(r0, _RPT)])

    @pl.when(s == _NS - 1)
    def _init_last():
        pltpu.sync_copy(init_hbm.at[c, pl.ds(15 * _RPT, _RLAST)],
                        acc_sh.at[pl.ds(15 * _RPT, _RLAST)])

    pltpu.sync_copy(idxs_hbm.at[c, s], idx_v)
    plsc.subcore_barrier()
    ebase = (c * _NS + s) * _SPW
    pltpu.async_copy(m_hbm.at[pl.ds(ebase, _K2)], mrows_v.at[0], msem)

    def chunk(g, carry):
        b = lax.rem(g, 2)
        pltpu.make_async_copy(m_hbm.at[pl.ds(0, _K2)], mrows_v.at[b],
                              msem).wait()

        @pl.when(g < _SC2 - 1)
        def _prefetch():
            off = pl.multiple_of(ebase + (g + 1) * _K2, 8)
            pltpu.async_copy(m_hbm.at[pl.ds(off, _K2)],
                             mrows_v.at[lax.rem(g + 1, 2)], msem)

        pltpu.sync_copy(mrows_v.at[b], acc_sh.at[idx_v.at[g]], add=True)
        return carry

    lax.fori_loop(0, _SC2, chunk, 0)
    plsc.subcore_barrier()

    @pl.when(s < _NS - 1)
    def _dump_main():
        pltpu.sync_copy(acc_sh.at[pl.ds(r0, _RPT)],
                        part_hbm.at[c, pl.ds(r0, _RPT)])

    @pl.when(s == _NS - 1)
    def _dump_last():
        pltpu.sync_copy(acc_sh.at[pl.ds(15 * _RPT, _RLAST)],
                        part_hbm.at[c, pl.ds(15 * _RPT, _RLAST)])


_scatter = pl.kernel(
    _scatter_body,
    out_type=jax.ShapeDtypeStruct((_NC, _N, _D), jnp.float32),
    mesh=_sc_mesh,
    scratch_types=[
        pltpu.VMEM((_SC2, _K2), jnp.int32),
        pltpu.VMEM((2, _K2, _D), jnp.float32),
        pltpu.SemaphoreType.DMA,
        pltpu.VMEM_SHARED((_N, _D), jnp.float32),
    ],
)


def _enc_body(nf, Wn, bn, out):
    out[...] = jnp.maximum(nf[...] @ Wn[...] + bn[...], 0.0)


def _encode_h(nf, Wn, bn2):
    blk = 2000
    return pl.pallas_call(
        _enc_body,
        grid=(_N // blk,),
        in_specs=[pl.BlockSpec((blk, _D), lambda i: (i, 0)),
                  pl.BlockSpec((_D, _D), lambda i: (0, 0)),
                  pl.BlockSpec((1, _D), lambda i: (0, 0))],
        out_specs=pl.BlockSpec((blk, _D), lambda i: (i, 0)),
        out_shape=jax.ShapeDtypeStruct((_N, _D), jnp.float32),
    )(nf, Wn, bn2)


_DNN = (((1,), (0,)), ((), ()))


def _msg_body(st, ef, A, B, C, We, be, bm1, Wm2, bm2, out):
    e = jnp.maximum(ef[...] @ We[...] + be[...], 0.0)
    stv = st[...].astype(jnp.bfloat16)
    m1 = (lax.dot_general(stv[0], A[...], _DNN,
                          preferred_element_type=jnp.float32)
          + lax.dot_general(stv[1], B[...], _DNN,
                            preferred_element_type=jnp.float32)
          + e @ C[...] + bm1[...])
    out[...] = lax.dot_general(jnp.maximum(m1, 0.0).astype(jnp.bfloat16),
                               Wm2[...], _DNN,
                               preferred_element_type=jnp.float32) + bm2[...]


def _messages(st, ef, A, B, C, We, be2, bm12, Wm2, bm22):
    blk = 4000
    return pl.pallas_call(
        _msg_body,
        grid=(_EH // blk,),
        in_specs=[pl.BlockSpec((_NC, blk, _D), lambda i: (0, i, 0)),
                  pl.BlockSpec((blk, _DE), lambda i: (i, 0)),
                  pl.BlockSpec((_D, _HID), lambda i: (0, 0)),
                  pl.BlockSpec((_D, _HID), lambda i: (0, 0)),
                  pl.BlockSpec((_DEH, _HID), lambda i: (0, 0)),
                  pl.BlockSpec((_DE, _DEH), lambda i: (0, 0)),
                  pl.BlockSpec((1, _DEH), lambda i: (0, 0)),
                  pl.BlockSpec((1, _HID), lambda i: (0, 0)),
                  pl.BlockSpec((_HID, _D), lambda i: (0, 0)),
                  pl.BlockSpec((1, _D), lambda i: (0, 0))],
        out_specs=pl.BlockSpec((blk, _D), lambda i: (i, 0)),
        out_shape=jax.ShapeDtypeStruct((_EH, _D), jnp.float32),
    )(st, ef, A, B, C, We, be2, bm12, Wm2, bm22)


def _rsm(x):
    m = jnp.max(x, axis=1, keepdims=True)
    ex = jnp.exp(x - m)
    return ex / jnp.sum(ex, axis=1, keepdims=True)


_PPB = 5


def _upd_body(hb, pb, U1h, U1a, U1c, bu1, Wu2, bu2, out):
    h = hb[...]
    p = pb[...]
    agg = p[0] + p[1]
    dnt = (((1,), (1,)), ((), ()))
    crosses = []
    for q in range(_PPB):
        a = h[2 * _G * q:2 * _G * q + _G]
        b = h[2 * _G * q + _G:2 * _G * (q + 1)]
        sim = lax.dot_general(a, b, dnt)
        simt = lax.dot_general(b, a, dnt)
        att_a = _rsm(sim) @ b
        att_b = _rsm(simt) @ a
        crosses.append(a - att_a)
        crosses.append(b - att_b)
    cross = jnp.concatenate(crosses, axis=0)
    u = jnp.maximum(h @ U1h[...] + agg @ U1a[...] + cross @ U1c[...]
                    + bu1[...], 0.0)
    out[...] = u @ Wu2[...] + bu2[...]


def _update(h, parts, U1h, U1a, U1c, bu12, Wu2, bu22):
    blk = 2 * _G * _PPB
    return pl.pallas_call(
        _upd_body,
        grid=(_NPAIRS // _PPB,),
        in_specs=[pl.BlockSpec((blk, _D), lambda i: (i, 0)),
                  pl.BlockSpec((_NC, blk, _D), lambda i: (0, i, 0)),
                  pl.BlockSpec((_D, _HID), lambda i: (0, 0)),
                  pl.BlockSpec((_D, _HID), lambda i: (0, 0)),
                  pl.BlockSpec((_D, _HID), lambda i: (0, 0)),
                  pl.BlockSpec((1, _HID), lambda i: (0, 0)),
                  pl.BlockSpec((_HID, _D), lambda i: (0, 0)),
                  pl.BlockSpec((1, _D), lambda i: (0, 0))],
        out_specs=pl.BlockSpec((blk, _D), lambda i: (i, 0)),
        out_shape=jax.ShapeDtypeStruct((_N, _D), jnp.float32),
    )(h, parts, U1h, U1a, U1c, bu12, Wu2, bu22)


def _pool_body(hb, Wg, bg, Wf, bf, out):
    gv = hb[...] @ Wg[...] + bg[...]
    gates = 1.0 / (1.0 + jnp.exp(-gv[:, :_D]))
    gated = gates * gv[:, _D:]
    sums = []
    for q in range(_PPB):
        sums.append(jnp.sum(gated[2 * _G * q:2 * _G * q + _G], axis=0,
                            keepdims=True))
    for q in range(_PPB):
        sums.append(jnp.sum(gated[2 * _G * q + _G:2 * _G * (q + 1)], axis=0,
                            keepdims=True))
    gs = jnp.concatenate(sums, axis=0)
    gvec = gs @ Wf[...] + bf[...]
    va = gvec[:_PPB]
    vb = gvec[_PPB:]
    r = jnp.maximum(va - vb, 0.0)
    sc = -jnp.sum(r, axis=1, keepdims=True)
    out[...] = jnp.broadcast_to(sc.reshape(_PPB, 1, 1), (_PPB, 1, _D))


def _pool(h, Wg, bg2, Wf, bf2):
    blk = 2 * _G * _PPB
    return pl.pallas_call(
        _pool_body,
        grid=(_NPAIRS // _PPB,),
        in_specs=[pl.BlockSpec((blk, _D), lambda i: (i, 0)),
                  pl.BlockSpec((_D, 2 * _D), lambda i: (0, 0)),
                  pl.BlockSpec((1, 2 * _D), lambda i: (0, 0)),
                  pl.BlockSpec((_D, _D), lambda i: (0, 0)),
                  pl.BlockSpec((1, _D), lambda i: (0, 0))],
        out_specs=pl.BlockSpec((_PPB, 1, _D), lambda i: (i, 0, 0)),
        out_shape=jax.ShapeDtypeStruct((_NPAIRS, 1, _D), jnp.float32),
    )(h, Wg, bg2, Wf, bf2)


def kernel(node_features, edge_features, from_idx, to_idx, graph_idx,
           Wn, bn, We, be, Wm1, bm1, Wm2, bm2, Wu1, bu1, Wu2, bu2,
           Wg, bg, Wf, bf):
    bn2 = bn.reshape(1, -1)
    be2 = be.reshape(1, -1)
    bm12 = bm1.reshape(1, -1)
    bm22 = bm2.reshape(1, -1)
    bu12 = bu1.reshape(1, -1)
    bu22 = bu2.reshape(1, -1)
    bg2 = bg.reshape(1, -1)
    bf2 = bf.reshape(1, -1)
    A = Wm1[:_D].astype(jnp.bfloat16)
    B = Wm1[_D:2 * _D].astype(jnp.bfloat16)
    C = Wm1[2 * _D:]
    U1h = Wu1[:_D]
    U1a = Wu1[_D:2 * _D]
    U1c = Wu1[2 * _D:]
    Wm2b = Wm2.astype(jnp.bfloat16)
    fi = from_idx.astype(jnp.int32)
    ti = to_idx.astype(jnp.int32)
    idxg1 = jnp.stack([fi[:_EH].reshape(_NS, _CPW, _KCH),
                       ti[:_EH].reshape(_NS, _CPW, _KCH)])
    idxg2 = jnp.stack([fi[_EH:].reshape(_NS, _CPW, _KCH),
                       ti[_EH:].reshape(_NS, _CPW, _KCH)])
    idxs1 = ti[:_EH].reshape(_NC, _NS, _SC2, _K2)
    idxs2 = ti[_EH:].reshape(_NC, _NS, _SC2, _K2)
    ef1 = edge_features[:_EH]
    ef2 = edge_features[_EH:]
    zeros = jnp.zeros((_NC, _N, _D), jnp.float32)

    h = _encode_h(node_features, Wn, bn2)
    for _ in range(_NPROP):
        st1 = _gather(h, idxg1)
        m1 = _messages(st1, ef1, A, B, C, We, be2, bm12, Wm2b, bm22)
        st2 = _gather(h, idxg2)
        m2 = _messages(st2, ef2, A, B, C, We, be2, bm12, Wm2b, bm22)
        parts1 = _scatter(m1, idxs1, zeros)
        parts = _scatter(m2, idxs2, parts1)
        h = _update(h, parts, U1h, U1a, U1c, bu12, Wu2, bu22)
    out = _pool(h, Wg, bg2, Wf, bf2)
    return out[:, 0, 0]
